# baseline (speedup 1.0000x reference)
import jax
import jax.numpy as jnp
from jax.experimental import pallas as pl
from jax.experimental.pallas import tpu as pltpu

S = 2048
D_MODEL = 1024
D_SPACE = 128
N_NEURONS = 4096
N_CLUSTERS = 64
CLUSTER_SIZE = N_NEURONS // N_CLUSTERS
MAX_K = 64
T_BLK = 256
GRID = S // T_BLK


def _fused_body(x_ref, proj_ref, pb_ref, tauk_ref, taub_ref, cemb_ref,
                nemb_ref, kn_ref, fold_ref, out_ref, cfreq_ref, nfreq_ref,
                caux_ref, naux_ref, nen_ref, cid_ref):
    i = pl.program_id(0)

    @pl.when(i == 0)
    def _():
        ne = nemb_ref[...]
        inv_n = 1.0 / (jnp.sqrt(jnp.sum(ne * ne, axis=-1, keepdims=True))
                       + 1e-08)
        nen_ref[...] = ne * inv_n
        lane = jax.lax.broadcasted_iota(jnp.int32, (1, N_NEURONS), 1)
        cid_ref[...] = jax.lax.shift_right_logical(lane, 6)
        cfreq_ref[...] = jnp.zeros_like(cfreq_ref)
        nfreq_ref[...] = jnp.zeros_like(nfreq_ref)

    xb = x_ref[...]
    hb = jnp.dot(xb, proj_ref[...],
                 preferred_element_type=jnp.float32) + pb_ref[...]
    taub = jnp.sum(xb * tauk_ref[...], axis=-1, keepdims=True) \
        + taub_ref[...]

    ce = cemb_ref[...]
    ce_n = ce / (jnp.sqrt(jnp.sum(ce * ce, axis=-1, keepdims=True)) + 1e-08)
    cs = jax.lax.dot_general(hb, ce_n, (((1,), (1,)), ((), ())),
                             preferred_element_type=jnp.float32)
    m = jnp.max(cs, axis=-1, keepdims=True)
    p = jnp.exp(cs - m)
    p = p / jnp.sum(p, axis=-1, keepdims=True)
    cfreq_ref[...] += jnp.sum(p, axis=0, keepdims=True)

    lane64 = jax.lax.broadcasted_iota(jnp.int32, (T_BLK, N_CLUSTERS), 1)
    big = jnp.int32(N_CLUSTERS + 1)
    a1 = jnp.min(jnp.where(cs == m, lane64, big), axis=-1, keepdims=True)
    cs2 = jnp.where(lane64 == a1, -jnp.inf, cs)
    m2 = jnp.max(cs2, axis=-1, keepdims=True)
    a2 = jnp.min(jnp.where(cs2 == m2, lane64, big), axis=-1, keepdims=True)

    s_all = jax.lax.dot_general(hb, nen_ref[...], (((1,), (1,)), ((), ())),
                                preferred_element_type=jnp.float32)

    cid = cid_ref[...]
    m1 = cid == a1
    m2m = cid == a2
    fold = fold_ref[...]
    b1 = jnp.where(m1, s_all, 0.0)
    b2 = jnp.where(m2m, s_all, 0.0)
    g1s = jax.lax.dot_general(b1, fold, (((1,), (0,)), ((), ())),
                              preferred_element_type=jnp.float32)
    g2s = jax.lax.dot_general(b2, fold, (((1,), (0,)), ((), ())),
                              preferred_element_type=jnp.float32)
    a_sc = jnp.concatenate([g1s, g2s], axis=1)

    raw_g = a_sc - taub
    gate_g = jnp.where(raw_g > 0, raw_g, 1e-08 * jnp.exp(raw_g))
    e_g = jnp.exp(gate_g) - 1.0

    bits = jax.lax.bitcast_convert_type(e_g, jnp.int32)
    thr_bits = jnp.zeros((T_BLK, 1), jnp.int32)
    for b in range(30, -1, -1):
        cand = thr_bits | jnp.int32(1 << b)
        cnt = jnp.sum((bits >= cand).astype(jnp.int32), axis=-1, keepdims=True)
        thr_bits = jnp.where(cnt >= MAX_K, cand, thr_bits)
    thr = jax.lax.bitcast_convert_type(thr_bits, jnp.float32)

    e_kept = jnp.where(e_g >= thr, e_g, 0.0)
    gsum = jnp.sum(e_kept, axis=-1, keepdims=True) + 1e-08
    gstr = jnp.tanh(jnp.max(e_kept, axis=-1, keepdims=True))
    g_val = e_kept * (gstr / gsum)

    g1 = g_val[:, :CLUSTER_SIZE]
    g2 = g_val[:, CLUSTER_SIZE:]

    p1 = jnp.where(lane64 == a1, 1.0, 0.0)
    p2 = jnp.where(lane64 == a2, 1.0, 0.0)
    nfreq_ref[...] += (
        jax.lax.dot_general(p1, g1, (((0,), (0,)), ((), ())),
                            preferred_element_type=jnp.float32)
        + jax.lax.dot_general(p2, g2, (((0,), (0,)), ((), ())),
                              preferred_element_type=jnp.float32))

    g1x = jax.lax.dot_general(g1, fold, (((1,), (1,)), ((), ())),
                              preferred_element_type=jnp.float32)
    g2x = jax.lax.dot_general(g2, fold, (((1,), (1,)), ((), ())),
                              preferred_element_type=jnp.float32)
    kn = kn_ref[...]
    act = jax.lax.dot_general(xb.astype(jnp.bfloat16), kn,
                              (((1,), (1,)), ((), ())),
                              preferred_element_type=jnp.float32)
    gated = act * (jnp.where(m1, g1x, 0.0) + jnp.where(m2m, g2x, 0.0))
    out_ref[...] = jnp.dot(gated.astype(jnp.bfloat16), kn,
                           preferred_element_type=jnp.float32)

    @pl.when(i == GRID - 1)
    def _():
        cfreq = cfreq_ref[...] * (1.0 / S)
        caux_ref[...] = jnp.sum((cfreq - 1.0 / N_CLUSTERS) ** 2,
                                keepdims=True) * N_CLUSTERS
        nfreq = nfreq_ref[...] * (1.0 / S)
        naux_ref[...] = jnp.sum((nfreq - 1.0 / N_NEURONS) ** 2,
                                keepdims=True)[:1, :1] * N_NEURONS


@jax.jit
def kernel(x, proj_kernel, proj_bias, tau_kernel, tau_bias,
           neuron_emb, cluster_emb, know_neurons):
    x2d = x.reshape(S, D_MODEL)
    fold = (jnp.arange(N_NEURONS, dtype=jnp.int32)[:, None] % CLUSTER_SIZE
            == jnp.arange(CLUSTER_SIZE, dtype=jnp.int32)[None, :]
            ).astype(jnp.float32)
    in_specs = [
            pl.BlockSpec((T_BLK, D_MODEL), lambda i: (i, 0)),
            pl.BlockSpec((D_MODEL, D_SPACE), lambda i: (0, 0)),
            pl.BlockSpec((1, D_SPACE), lambda i: (0, 0)),
            pl.BlockSpec((1, D_MODEL), lambda i: (0, 0)),
            pl.BlockSpec((1, 1), lambda i: (0, 0)),
            pl.BlockSpec((N_CLUSTERS, D_SPACE), lambda i: (0, 0)),
            pl.BlockSpec((N_NEURONS, D_SPACE), lambda i: (0, 0)),
            pl.BlockSpec((N_NEURONS, D_MODEL), lambda i: (0, 0)),
            pl.BlockSpec((N_NEURONS, CLUSTER_SIZE), lambda i: (0, 0)),
    ]
    out_specs = [
            pl.BlockSpec((T_BLK, D_MODEL), lambda i: (i, 0)),
            pl.BlockSpec((1, N_CLUSTERS), lambda i: (0, 0)),
            pl.BlockSpec((N_CLUSTERS, CLUSTER_SIZE), lambda i: (0, 0)),
            pl.BlockSpec((1, 1), lambda i: (0, 0)),
            pl.BlockSpec((1, 1), lambda i: (0, 0)),
    ]
    out, _, _, caux, naux = pl.pallas_call(
        _fused_body,
        grid=(GRID,),
        in_specs=in_specs,
        out_specs=out_specs,
        out_shape=[
            jax.ShapeDtypeStruct((S, D_MODEL), jnp.float32),
            jax.ShapeDtypeStruct((1, N_CLUSTERS), jnp.float32),
            jax.ShapeDtypeStruct((N_CLUSTERS, CLUSTER_SIZE), jnp.float32),
            jax.ShapeDtypeStruct((1, 1), jnp.float32),
            jax.ShapeDtypeStruct((1, 1), jnp.float32),
        ],
        scratch_shapes=[
            pltpu.VMEM((N_NEURONS, D_SPACE), jnp.float32),
            pltpu.VMEM((1, N_NEURONS), jnp.int32),
        ],
        compiler_params=pltpu.CompilerParams(
            dimension_semantics=("arbitrary",),
        ),
    )(x2d, proj_kernel, proj_bias.reshape(1, D_SPACE),
      tau_kernel.reshape(1, D_MODEL), tau_bias.reshape(1, 1),
      cluster_emb, neuron_emb, know_neurons.astype(jnp.bfloat16), fold)
    return (out.reshape(1, S, D_MODEL), caux.reshape(()), naux.reshape(()))


# speedup vs baseline: 1.0559x; 1.0559x over previous
import jax
import jax.numpy as jnp
from jax.experimental import pallas as pl
from jax.experimental.pallas import tpu as pltpu

S = 2048
D_MODEL = 1024
D_SPACE = 128
N_NEURONS = 4096
N_CLUSTERS = 64
CLUSTER_SIZE = N_NEURONS // N_CLUSTERS
MAX_K = 64
T_BLK = 256
GRID = S // T_BLK


def _fused_body(x_ref, proj_ref, pb_ref, tauk_ref, taub_ref, cemb_ref,
                nemb_ref, kn_ref, fold_ref, out_ref, cfreq_ref, nfreq_ref,
                caux_ref, naux_ref, nen_ref, cid_ref, gbuf_ref):
    i = pl.program_id(0)

    @pl.when(i == 0)
    def _():
        ne = nemb_ref[...]
        inv_n = 1.0 / (jnp.sqrt(jnp.sum(ne * ne, axis=-1, keepdims=True))
                       + 1e-08)
        nen_ref[...] = ne * inv_n
        lane = jax.lax.broadcasted_iota(jnp.int32, (1, N_NEURONS), 1)
        cid_ref[...] = jax.lax.shift_right_logical(lane, 6)
        cfreq_ref[...] = jnp.zeros_like(cfreq_ref)
        nfreq_ref[...] = jnp.zeros_like(nfreq_ref)

    kn = kn_ref[...]

    @pl.when(i > 0)
    def _():
        out_ref[...] = jnp.dot(gbuf_ref[jax.lax.rem(i - 1, 2)], kn,
                               preferred_element_type=jnp.float32)

    @pl.when(i < GRID)
    def _():
        _routing_phase(i, x_ref, proj_ref, pb_ref, tauk_ref, taub_ref,
                       cemb_ref, kn, fold_ref, cfreq_ref, nfreq_ref,
                       nen_ref, cid_ref, gbuf_ref)

    @pl.when(i == GRID - 1)
    def _():
        cfreq = cfreq_ref[...] * (1.0 / S)
        caux_ref[...] = jnp.sum((cfreq - 1.0 / N_CLUSTERS) ** 2,
                                keepdims=True) * N_CLUSTERS
        nfreq = nfreq_ref[...] * (1.0 / S)
        naux_ref[...] = jnp.sum((nfreq - 1.0 / N_NEURONS) ** 2,
                                keepdims=True)[:1, :1] * N_NEURONS


def _routing_phase(i, x_ref, proj_ref, pb_ref, tauk_ref, taub_ref, cemb_ref,
                   kn, fold_ref, cfreq_ref, nfreq_ref, nen_ref, cid_ref,
                   gbuf_ref):
    xb = x_ref[...]
    hb = jnp.dot(xb, proj_ref[...],
                 preferred_element_type=jnp.float32) + pb_ref[...]
    taub = jnp.sum(xb * tauk_ref[...], axis=-1, keepdims=True) \
        + taub_ref[...]

    act = jax.lax.dot_general(xb, kn, (((1,), (1,)), ((), ())),
                              preferred_element_type=jnp.float32)

    ce = cemb_ref[...]
    ce_n = ce / (jnp.sqrt(jnp.sum(ce * ce, axis=-1, keepdims=True)) + 1e-08)
    cs = jax.lax.dot_general(hb, ce_n, (((1,), (1,)), ((), ())),
                             preferred_element_type=jnp.float32)
    m = jnp.max(cs, axis=-1, keepdims=True)
    p = jnp.exp(cs - m)
    p = p / jnp.sum(p, axis=-1, keepdims=True)
    cfreq_ref[...] += jnp.sum(p, axis=0, keepdims=True)

    lane64 = jax.lax.broadcasted_iota(jnp.int32, (T_BLK, N_CLUSTERS), 1)
    big = jnp.int32(N_CLUSTERS + 1)
    a1 = jnp.min(jnp.where(cs == m, lane64, big), axis=-1, keepdims=True)
    cs2 = jnp.where(lane64 == a1, -jnp.inf, cs)
    m2 = jnp.max(cs2, axis=-1, keepdims=True)
    a2 = jnp.min(jnp.where(cs2 == m2, lane64, big), axis=-1, keepdims=True)

    s_all = jax.lax.dot_general(hb, nen_ref[...], (((1,), (1,)), ((), ())),
                                preferred_element_type=jnp.float32)

    cid = cid_ref[...]
    m1 = cid == a1
    m2m = cid == a2
    fold = fold_ref[...]
    b1 = jnp.where(m1, s_all, 0.0)
    b2 = jnp.where(m2m, s_all, 0.0)
    g1s = jax.lax.dot_general(b1, fold, (((1,), (0,)), ((), ())),
                              preferred_element_type=jnp.float32)
    g2s = jax.lax.dot_general(b2, fold, (((1,), (0,)), ((), ())),
                              preferred_element_type=jnp.float32)
    a_sc = jnp.concatenate([g1s, g2s], axis=1)

    raw_g = a_sc - taub
    gate_g = jnp.where(raw_g > 0, raw_g, 1e-08 * jnp.exp(raw_g))
    e_g = jnp.exp(gate_g) - 1.0

    bits = jax.lax.bitcast_convert_type(e_g, jnp.int32)
    thr_bits = jnp.zeros((T_BLK, 1), jnp.int32)

    def _ge_count(cand):
        return jnp.sum(jnp.where(bits >= cand, 1.0, 0.0),
                       axis=-1, keepdims=True) >= float(MAX_K)

    for b in range(30, 0, -2):
        hi, lo = jnp.int32(1 << b), jnp.int32(1 << (b - 1))
        c_a = thr_bits | hi
        c_ab = c_a | lo
        c_b = thr_bits | lo
        n_a, n_ab, n_b = _ge_count(c_a), _ge_count(c_ab), _ge_count(c_b)
        thr_bits = jnp.where(n_a, jnp.where(n_ab, c_ab, c_a),
                             jnp.where(n_b, c_b, thr_bits))
    c0 = thr_bits | jnp.int32(1)
    thr_bits = jnp.where(_ge_count(c0), c0, thr_bits)
    thr = jax.lax.bitcast_convert_type(thr_bits, jnp.float32)

    e_kept = jnp.where(e_g >= thr, e_g, 0.0)
    gsum = jnp.sum(e_kept, axis=-1, keepdims=True) + 1e-08
    gstr = jnp.tanh(jnp.max(e_kept, axis=-1, keepdims=True))
    g_val = e_kept * (gstr / gsum)

    g1 = g_val[:, :CLUSTER_SIZE]
    g2 = g_val[:, CLUSTER_SIZE:]

    p1 = jnp.where(lane64 == a1, 1.0, 0.0)
    p2 = jnp.where(lane64 == a2, 1.0, 0.0)
    nfreq_ref[...] += (
        jax.lax.dot_general(p1, g1, (((0,), (0,)), ((), ())),
                            preferred_element_type=jnp.float32)
        + jax.lax.dot_general(p2, g2, (((0,), (0,)), ((), ())),
                              preferred_element_type=jnp.float32))

    g1x = jax.lax.dot_general(g1, fold, (((1,), (1,)), ((), ())),
                              preferred_element_type=jnp.float32)
    g2x = jax.lax.dot_general(g2, fold, (((1,), (1,)), ((), ())),
                              preferred_element_type=jnp.float32)
    gbuf_ref[jax.lax.rem(i, 2)] = act * jnp.where(m1, g1x,
                                                  jnp.where(m2m, g2x, 0.0))


@jax.jit
def kernel(x, proj_kernel, proj_bias, tau_kernel, tau_bias,
           neuron_emb, cluster_emb, know_neurons):
    x2d = x.reshape(S, D_MODEL)
    fold = (jnp.arange(N_NEURONS, dtype=jnp.int32)[:, None] % CLUSTER_SIZE
            == jnp.arange(CLUSTER_SIZE, dtype=jnp.int32)[None, :]
            ).astype(jnp.float32)
    in_specs = [
            pl.BlockSpec((T_BLK, D_MODEL),
                         lambda i: (jnp.minimum(i, GRID - 1), 0)),
            pl.BlockSpec((D_MODEL, D_SPACE), lambda i: (0, 0)),
            pl.BlockSpec((1, D_SPACE), lambda i: (0, 0)),
            pl.BlockSpec((1, D_MODEL), lambda i: (0, 0)),
            pl.BlockSpec((1, 1), lambda i: (0, 0)),
            pl.BlockSpec((N_CLUSTERS, D_SPACE), lambda i: (0, 0)),
            pl.BlockSpec((N_NEURONS, D_SPACE), lambda i: (0, 0)),
            pl.BlockSpec((N_NEURONS, D_MODEL), lambda i: (0, 0)),
            pl.BlockSpec((N_NEURONS, CLUSTER_SIZE), lambda i: (0, 0)),
    ]
    out_specs = [
            pl.BlockSpec((T_BLK, D_MODEL),
                         lambda i: (jnp.maximum(i - 1, 0), 0)),
            pl.BlockSpec((1, N_CLUSTERS), lambda i: (0, 0)),
            pl.BlockSpec((N_CLUSTERS, CLUSTER_SIZE), lambda i: (0, 0)),
            pl.BlockSpec((1, 1), lambda i: (0, 0)),
            pl.BlockSpec((1, 1), lambda i: (0, 0)),
    ]
    out, _, _, caux, naux = pl.pallas_call(
        _fused_body,
        grid=(GRID + 1,),
        in_specs=in_specs,
        out_specs=out_specs,
        out_shape=[
            jax.ShapeDtypeStruct((S, D_MODEL), jnp.float32),
            jax.ShapeDtypeStruct((1, N_CLUSTERS), jnp.float32),
            jax.ShapeDtypeStruct((N_CLUSTERS, CLUSTER_SIZE), jnp.float32),
            jax.ShapeDtypeStruct((1, 1), jnp.float32),
            jax.ShapeDtypeStruct((1, 1), jnp.float32),
        ],
        scratch_shapes=[
            pltpu.VMEM((N_NEURONS, D_SPACE), jnp.float32),
            pltpu.VMEM((1, N_NEURONS), jnp.int32),
            pltpu.VMEM((2, T_BLK, N_NEURONS), jnp.float32),
        ],
        compiler_params=pltpu.CompilerParams(
            dimension_semantics=("arbitrary",),
            vmem_limit_bytes=100 * 1024 * 1024,
        ),
    )(x2d, proj_kernel, proj_bias.reshape(1, D_SPACE),
      tau_kernel.reshape(1, D_MODEL), tau_bias.reshape(1, 1),
      cluster_emb, neuron_emb, know_neurons, fold)
    return (out.reshape(1, S, D_MODEL), caux.reshape(()), naux.reshape(()))


# speedup vs baseline: 1.1773x; 1.1150x over previous
import jax
import jax.numpy as jnp
from jax.experimental import pallas as pl
from jax.experimental.pallas import tpu as pltpu

S = 2048
D_MODEL = 1024
D_SPACE = 128
N_NEURONS = 4096
N_CLUSTERS = 64
CLUSTER_SIZE = N_NEURONS // N_CLUSTERS
MAX_K = 64
T_BLK = 256
GRID = S // T_BLK


def _fused_body(x_ref, proj_ref, pb_ref, tauk_ref, taub_ref, cemb_ref,
                nemb_ref, kn_ref, fold_ref, out_ref, cfreq_ref, nfreq_ref,
                caux_ref, naux_ref, nen_ref, cid_ref):
    i = pl.program_id(0)

    @pl.when(i == 0)
    def _():
        ne = nemb_ref[...]
        inv_n = 1.0 / (jnp.sqrt(jnp.sum(ne * ne, axis=-1, keepdims=True))
                       + 1e-08)
        nen_ref[...] = ne * inv_n
        lane = jax.lax.broadcasted_iota(jnp.int32, (1, N_NEURONS), 1)
        cid_ref[...] = jax.lax.shift_right_logical(lane, 6)
        cfreq_ref[...] = jnp.zeros_like(cfreq_ref)
        nfreq_ref[...] = jnp.zeros_like(nfreq_ref)

    xb = x_ref[...]
    hb = jnp.dot(xb, proj_ref[...],
                 preferred_element_type=jnp.float32) + pb_ref[...]
    taub = jnp.sum(xb * tauk_ref[...], axis=-1, keepdims=True) \
        + taub_ref[...]

    kn = kn_ref[...]
    act = jax.lax.dot_general(xb, kn, (((1,), (1,)), ((), ())),
                              preferred_element_type=jnp.float32)

    ce = cemb_ref[...]
    ce_n = ce / (jnp.sqrt(jnp.sum(ce * ce, axis=-1, keepdims=True)) + 1e-08)
    cs = jax.lax.dot_general(hb, ce_n, (((1,), (1,)), ((), ())),
                             preferred_element_type=jnp.float32)
    m = jnp.max(cs, axis=-1, keepdims=True)
    p = jnp.exp(cs - m)
    p = p / jnp.sum(p, axis=-1, keepdims=True)
    cfreq_ref[...] += jnp.sum(p, axis=0, keepdims=True)

    lane64 = jax.lax.broadcasted_iota(jnp.int32, (T_BLK, N_CLUSTERS), 1)
    big = jnp.int32(N_CLUSTERS + 1)
    a1 = jnp.min(jnp.where(cs == m, lane64, big), axis=-1, keepdims=True)
    cs2 = jnp.where(lane64 == a1, -jnp.inf, cs)
    m2 = jnp.max(cs2, axis=-1, keepdims=True)
    a2 = jnp.min(jnp.where(cs2 == m2, lane64, big), axis=-1, keepdims=True)

    s_all = jax.lax.dot_general(hb, nen_ref[...], (((1,), (1,)), ((), ())),
                                preferred_element_type=jnp.float32)

    cid = cid_ref[...]
    m1 = cid == a1
    m2m = cid == a2
    fold = fold_ref[...]
    b1 = jnp.where(m1, s_all, 0.0)
    b2 = jnp.where(m2m, s_all, 0.0)
    g1s = jax.lax.dot_general(b1, fold, (((1,), (0,)), ((), ())),
                              preferred_element_type=jnp.float32)
    g2s = jax.lax.dot_general(b2, fold, (((1,), (0,)), ((), ())),
                              preferred_element_type=jnp.float32)
    a_sc = jnp.concatenate([g1s, g2s], axis=1)

    raw_g = a_sc - taub
    gate_g = jnp.where(raw_g > 0, raw_g, 1e-08 * jnp.exp(raw_g))
    e_g = jnp.exp(gate_g) - 1.0

    bits = jax.lax.bitcast_convert_type(e_g, jnp.int32)
    thr_bits = jnp.zeros((T_BLK, 1), jnp.int32)

    def _ge_count(cand):
        return jnp.sum(jnp.where(bits >= cand, 1.0, 0.0),
                       axis=-1, keepdims=True) >= float(MAX_K)

    for b in range(30, 0, -2):
        hi, lo = jnp.int32(1 << b), jnp.int32(1 << (b - 1))
        c_a = thr_bits | hi
        c_ab = c_a | lo
        c_b = thr_bits | lo
        n_a, n_ab, n_b = _ge_count(c_a), _ge_count(c_ab), _ge_count(c_b)
        thr_bits = jnp.where(n_a, jnp.where(n_ab, c_ab, c_a),
                             jnp.where(n_b, c_b, thr_bits))
    c0 = thr_bits | jnp.int32(1)
    thr_bits = jnp.where(_ge_count(c0), c0, thr_bits)
    thr = jax.lax.bitcast_convert_type(thr_bits, jnp.float32)

    e_kept = jnp.where(e_g >= thr, e_g, 0.0)
    gsum = jnp.sum(e_kept, axis=-1, keepdims=True) + 1e-08
    gstr = jnp.tanh(jnp.max(e_kept, axis=-1, keepdims=True))
    g_val = e_kept * (gstr / gsum)

    g1 = g_val[:, :CLUSTER_SIZE]
    g2 = g_val[:, CLUSTER_SIZE:]

    p1 = jnp.where(lane64 == a1, 1.0, 0.0)
    p2 = jnp.where(lane64 == a2, 1.0, 0.0)
    nfreq_ref[...] += (
        jax.lax.dot_general(p1, g1, (((0,), (0,)), ((), ())),
                            preferred_element_type=jnp.float32)
        + jax.lax.dot_general(p2, g2, (((0,), (0,)), ((), ())),
                              preferred_element_type=jnp.float32))

    g1x = jax.lax.dot_general(g1, fold, (((1,), (1,)), ((), ())),
                              preferred_element_type=jnp.float32)
    g2x = jax.lax.dot_general(g2, fold, (((1,), (1,)), ((), ())),
                              preferred_element_type=jnp.float32)
    gated = act * jnp.where(m1, g1x, jnp.where(m2m, g2x, 0.0))
    out_ref[...] = jnp.dot(gated, kn, preferred_element_type=jnp.float32)

    @pl.when(i == GRID - 1)
    def _():
        cfreq = cfreq_ref[...] * (1.0 / S)
        caux_ref[...] = jnp.sum((cfreq - 1.0 / N_CLUSTERS) ** 2,
                                keepdims=True) * N_CLUSTERS
        nfreq = nfreq_ref[...] * (1.0 / S)
        naux_ref[...] = jnp.sum((nfreq - 1.0 / N_NEURONS) ** 2,
                                keepdims=True)[:1, :1] * N_NEURONS


@jax.jit
def kernel(x, proj_kernel, proj_bias, tau_kernel, tau_bias,
           neuron_emb, cluster_emb, know_neurons):
    x2d = x.reshape(S, D_MODEL)
    fold = (jnp.arange(N_NEURONS, dtype=jnp.int32)[:, None] % CLUSTER_SIZE
            == jnp.arange(CLUSTER_SIZE, dtype=jnp.int32)[None, :]
            ).astype(jnp.float32)
    in_specs = [
        pl.BlockSpec((T_BLK, D_MODEL), lambda i: (i, 0)),
        pl.BlockSpec((D_MODEL, D_SPACE), lambda i: (0, 0)),
        pl.BlockSpec((1, D_SPACE), lambda i: (0, 0)),
        pl.BlockSpec((1, D_MODEL), lambda i: (0, 0)),
        pl.BlockSpec((1, 1), lambda i: (0, 0)),
        pl.BlockSpec((N_CLUSTERS, D_SPACE), lambda i: (0, 0)),
        pl.BlockSpec((N_NEURONS, D_SPACE), lambda i: (0, 0)),
        pl.BlockSpec((N_NEURONS, D_MODEL), lambda i: (0, 0)),
        pl.BlockSpec((N_NEURONS, CLUSTER_SIZE), lambda i: (0, 0)),
    ]
    out_specs = [
        pl.BlockSpec((T_BLK, D_MODEL), lambda i: (i, 0)),
        pl.BlockSpec((1, N_CLUSTERS), lambda i: (0, 0)),
        pl.BlockSpec((N_CLUSTERS, CLUSTER_SIZE), lambda i: (0, 0)),
        pl.BlockSpec((1, 1), lambda i: (0, 0)),
        pl.BlockSpec((1, 1), lambda i: (0, 0)),
    ]
    out, _, _, caux, naux = pl.pallas_call(
        _fused_body,
        grid=(GRID,),
        in_specs=in_specs,
        out_specs=out_specs,
        out_shape=[
            jax.ShapeDtypeStruct((S, D_MODEL), jnp.float32),
            jax.ShapeDtypeStruct((1, N_CLUSTERS), jnp.float32),
            jax.ShapeDtypeStruct((N_CLUSTERS, CLUSTER_SIZE), jnp.float32),
            jax.ShapeDtypeStruct((1, 1), jnp.float32),
            jax.ShapeDtypeStruct((1, 1), jnp.float32),
        ],
        scratch_shapes=[
            pltpu.VMEM((N_NEURONS, D_SPACE), jnp.float32),
            pltpu.VMEM((1, N_NEURONS), jnp.int32),
        ],
        compiler_params=pltpu.CompilerParams(
            dimension_semantics=("arbitrary",),
            vmem_limit_bytes=100 * 1024 * 1024,
        ),
    )(x2d, proj_kernel, proj_bias.reshape(1, D_SPACE),
      tau_kernel.reshape(1, D_MODEL), tau_bias.reshape(1, 1),
      cluster_emb, neuron_emb, know_neurons, fold)
    return (out.reshape(1, S, D_MODEL), caux.reshape(()), naux.reshape(()))
